# baseline (speedup 1.0000x reference)
import functools
import jax
import jax.numpy as jnp
from jax import lax
from jax.experimental import pallas as pl
from jax.experimental.pallas import tpu as pltpu
from jax.experimental.pallas import tpu_sc as plsc

NC = 2
NS = 16
NW = NC * NS
L = 16
CH = 80


def _f32(*shape):
    return jax.ShapeDtypeStruct(shape, jnp.float32)


def _mesh():
    return plsc.VectorSubcoreMesh(core_axis_name="c", subcore_axis_name="s")


def _wid():
    return lax.axis_index("s") * NC + lax.axis_index("c")


def _col(c):
    return jnp.full((L,), c, jnp.int32)


def _lrelu(x):
    return jnp.where(x > 0, x, 0.01 * x)


def _tc0(h, w1, b1, w2a, w2b, f1, al, ar, u_ref, v_ref, z_ref, el_ref, er_ref):
    hv = h[...]
    hg = jnp.dot(hv, w1[...], preferred_element_type=jnp.float32) + b1[...]
    u_ref[...] = jnp.dot(hg, w2a[...], preferred_element_type=jnp.float32)
    v_ref[...] = jnp.dot(hg, w2b[...], preferred_element_type=jnp.float32)
    z = jnp.dot(hv, f1[...], preferred_element_type=jnp.float32)
    z_ref[...] = z
    el_ref[...] = jnp.dot(z, al[...], preferred_element_type=jnp.float32)
    er_ref[...] = jnp.dot(z, ar[...], preferred_element_type=jnp.float32)


def _tc1(acc, fc2, a2l, a2r, z2_ref, el2_ref, er2_ref):
    a = acc[0] + acc[1]
    num = a[:, :64]
    den = a[:, 64:68]
    den = jnp.where(den == 0.0, 1.0, den)
    n = num.shape[0]
    den_rep = jnp.concatenate(
        [jnp.broadcast_to(den[:, i:i + 1], (n, 16)) for i in range(4)], axis=1)
    h1 = _lrelu(num / den_rep)
    z2 = jnp.dot(h1, fc2[...], preferred_element_type=jnp.float32)
    z2_ref[...] = z2
    el2_ref[...] = jnp.dot(z2, a2l[...], preferred_element_type=jnp.float32)
    er2_ref[...] = jnp.dot(z2, a2r[...], preferred_element_type=jnp.float32)


def _tc2(acc, wpl, wpr, p_ref, q_ref):
    a = acc[0] + acc[1]
    den = a[:, 64:65]
    den = jnp.where(den == 0.0, 1.0, den)
    h2 = a[:, :64] / den
    p_ref[...] = jnp.dot(h2, wpl[...], preferred_element_type=jnp.float32)
    q_ref[...] = jnp.dot(h2, wpr[...], preferred_element_type=jnp.float32)


def _zero_lane16(buf, rows, c0):
    z = jnp.zeros((L,), jnp.float32)

    def body(r, _):
        buf[r, pl.ds(c0, L)] = z
        return 0

    lax.fori_loop(0, rows, body, 0)


def _make_sc1(E, n):
    EP = E // NW
    NCHUNK = EP // CH

    @functools.partial(
        pl.kernel,
        mesh=_mesh(),
        compiler_params=pltpu.CompilerParams(needs_layout_passes=False, use_tc_tiling_on_sc=False),
        out_type=[_f32(E), _f32(NW, L)],
        scratch_types=[
            pltpu.VMEM((CH,), jnp.int32),
            pltpu.VMEM((CH,), jnp.int32),
            pltpu.VMEM((CH,), jnp.int32),
            pltpu.VMEM((CH,), jnp.int32),
            pltpu.VMEM((CH, 16), jnp.float32),
            pltpu.VMEM((CH, 16), jnp.float32),
            pltpu.VMEM((CH, 16), jnp.float32),
            pltpu.VMEM((CH, 16), jnp.float32),
            pltpu.VMEM((CH,), jnp.float32),
            pltpu.VMEM((CH,), jnp.float32),
            pltpu.VMEM((32,), jnp.float32),
            pltpu.VMEM((L,), jnp.float32),
            pltpu.SemaphoreType.DMA,
            pltpu.SemaphoreType.DMA,
            pltpu.SemaphoreType.DMA,
            pltpu.SemaphoreType.DMA,
            pltpu.SemaphoreType.DMA,
            pltpu.SemaphoreType.DMA,
        ],
    )
    def sc1(src_h, dst_h, utab_h, dtab_h, prm_h, score_h, mm_h,
            idx_s0, idx_s1, idx_d0, idx_d1, urows0, urows1, drows0, drows1,
            sbuf0, sbuf1, prm_v, mmbuf,
            sem_i0, sem_i1, sem_g0, sem_g1, sem_t0, sem_t1):
        wid = _wid()
        base = wid * EP
        IS = [idx_s0, idx_s1]
        ID = [idx_d0, idx_d1]
        UR = [urows0, urows1]
        DR = [drows0, drows1]
        SB = [sbuf0, sbuf1]
        SI = [sem_i0, sem_i1]
        SG = [sem_g0, sem_g1]
        ST = [sem_t0, sem_t1]
        pltpu.sync_copy(prm_h, prm_v)
        pva = prm_v[pl.ds(0, L)]
        pvb = prm_v[pl.ds(L, L)]
        b2 = [pva[k] for k in range(8)]
        w3 = [pva[8 + k] for k in range(8)]
        b3 = pvb[0]

        def issue_idx(c, s):
            eb = base + c * CH
            pltpu.async_copy(src_h.at[pl.ds(eb, CH)], IS[s], SI[s])
            pltpu.async_copy(dst_h.at[pl.ds(eb, CH)], ID[s], SI[s])

        def drain_idx(c, s):
            eb = base + c * CH
            pltpu.make_async_copy(src_h.at[pl.ds(eb, CH)], IS[s],
                                  SI[s]).wait()
            pltpu.make_async_copy(dst_h.at[pl.ds(eb, CH)], ID[s],
                                  SI[s]).wait()

        def issue_gather(s):
            pltpu.async_copy(utab_h.at[IS[s]], UR[s], SG[s])
            pltpu.async_copy(dtab_h.at[ID[s]], DR[s], SG[s])

        def drain_gather(s):
            pltpu.make_async_copy(utab_h.at[IS[s]], UR[s], SG[s]).wait()
            pltpu.make_async_copy(dtab_h.at[ID[s]], DR[s], SG[s]).wait()

        def issue_store(c, s):
            eb = base + c * CH
            pltpu.async_copy(SB[s], score_h.at[pl.ds(eb, CH)], ST[s])

        def drain_store(c, s):
            eb = base + c * CH
            pltpu.make_async_copy(SB[s], score_h.at[pl.ds(eb, CH)],
                                  ST[s]).wait()

        def chunk(c, b, first, n1, n2, carry):
            vmin, vmax = carry
            drain_gather(b)
            if n1:
                drain_idx(c + 1, 1 - b)
                issue_gather(1 - b)
            if n2 == 'always':
                issue_idx(c + 2, b)
            elif n2 == 'cond':
                @pl.when(c + 2 < NCHUNK)
                def _():
                    issue_idx(c + 2, b)
            if not first:
                drain_store(c - 2, b)
            for g in range(CH // L):
                rows = lax.iota(jnp.int32, L) + g * L
                acc = jnp.full((L,), 0.0, jnp.float32) + b3
                for k in range(8):
                    uk = plsc.load_gather(UR[b], [rows, _col(k)])
                    vk = plsc.load_gather(DR[b], [rows, _col(k)])
                    s = jnp.maximum(uk + vk + b2[k], 0.0)
                    acc = acc + s * w3[k]
                SB[b][pl.ds(g * L, L)] = acc
                vmin = jnp.minimum(vmin, acc)
                vmax = jnp.maximum(vmax, acc)
            issue_store(c, b)
            return vmin, vmax

        init = (jnp.full((L,), jnp.inf, jnp.float32),
                jnp.full((L,), -jnp.inf, jnp.float32))
        issue_idx(0, 0)
        drain_idx(0, 0)
        issue_gather(0)
        issue_idx(1, 1)
        carry = chunk(0, 0, True, True, 'always', init)
        carry = chunk(1, 1, True, True, 'always', carry)

        def pair(j, carry):
            c0 = 2 * j
            carry = chunk(c0, 0, False, True, 'always', carry)
            carry = chunk(c0 + 1, 1, False, True, 'cond', carry)
            return carry

        carry = lax.fori_loop(1, (NCHUNK - 1) // 2, pair, carry)
        vmin, vmax = chunk(NCHUNK - 1, 0, False, False, 'no', carry)
        drain_store(NCHUNK - 2, 1)
        drain_store(NCHUNK - 1, 0)
        mn = jnp.min(vmin)
        mx = jnp.max(vmax)
        lane = lax.iota(jnp.int32, L)
        mmbuf[...] = jnp.where(lane == 0, mn, jnp.where(lane == 1, mx, 0.0))
        pltpu.sync_copy(mmbuf, mm_h.at[wid])

    return sc1


def _make_gat_pass(E, n, heads, store_gate):
    EP = E // NW
    NCHUNK = EP // CH
    NR = n // NS
    NRC = NR // 125

    @functools.partial(
        pl.kernel,
        mesh=_mesh(),
        compiler_params=pltpu.CompilerParams(needs_layout_passes=False, use_tc_tiling_on_sc=False),
        out_type=[_f32(E), _f32(NC, n, 80)],
        scratch_types=[
            pltpu.VMEM((CH,), jnp.int32),
            pltpu.VMEM((CH,), jnp.int32),
            pltpu.VMEM((CH,), jnp.int32),
            pltpu.VMEM((CH,), jnp.int32),
            pltpu.VMEM((CH, 80), jnp.float32),
            pltpu.VMEM((CH, 80), jnp.float32),
            pltpu.VMEM((CH, 16), jnp.float32),
            pltpu.VMEM((CH, 16), jnp.float32),
            pltpu.VMEM((CH,), jnp.float32),
            pltpu.VMEM((CH,), jnp.float32),
            pltpu.VMEM((CH,), jnp.float32),
            pltpu.VMEM((CH,), jnp.float32),
            pltpu.VMEM((CH, 80), jnp.float32),
            pltpu.VMEM((16,), jnp.float32),
            pltpu.VMEM((125, 80), jnp.float32),
            pltpu.VMEM_SHARED((n, 80), jnp.float32),
            pltpu.SemaphoreType.DMA,
            pltpu.SemaphoreType.DMA,
            pltpu.SemaphoreType.DMA,
            pltpu.SemaphoreType.DMA,
            pltpu.SemaphoreType.DMA,
            pltpu.SemaphoreType.DMA,
        ],
    )
    def scpass(src_h, dst_h, stab_h, dtab_h, gin_h, prm_h, gout_h, acc_h,
               idx_s0, idx_s1, idx_d0, idx_d1, srows0, srows1, drows0,
               drows1, gbuf0, gbuf1, gobuf0, gobuf1, obuf, prm_v, zb,
               shacc, sem_i0, sem_i1, sem_g0, sem_g1, sem_t0, sem_t1):
        cid = lax.axis_index("c")
        sid = lax.axis_index("s")
        wid = sid * NC + cid
        base = wid * EP
        IS = [idx_s0, idx_s1]
        ID = [idx_d0, idx_d1]
        SR = [srows0, srows1]
        DR = [drows0, drows1]
        GB = [gbuf0, gbuf1]
        GO = [gobuf0, gobuf1]
        SI = [sem_i0, sem_i1]
        SG = [sem_g0, sem_g1]
        ST = [sem_t0, sem_t1]

        pltpu.sync_copy(prm_h, prm_v)
        pv = prm_v[pl.ds(0, L)]
        mn = pv[0]
        gscale = pv[1]

        for c5 in range(5):
            _zero_lane16(zb, 125, c5 * L)
        for j in range(NRC):
            pltpu.sync_copy(zb, shacc.at[pl.ds(sid * NR + j * 125, 125)])
        _zero_lane16(obuf, CH, 64)
        plsc.subcore_barrier()

        def issue_idx(c, s):
            eb = base + c * CH
            pltpu.async_copy(src_h.at[pl.ds(eb, CH)], IS[s], SI[s])
            pltpu.async_copy(dst_h.at[pl.ds(eb, CH)], ID[s], SI[s])
            pltpu.async_copy(gin_h.at[pl.ds(eb, CH)], GB[s], SI[s])

        def drain_idx(c, s):
            eb = base + c * CH
            pltpu.make_async_copy(src_h.at[pl.ds(eb, CH)], IS[s],
                                  SI[s]).wait()
            pltpu.make_async_copy(dst_h.at[pl.ds(eb, CH)], ID[s],
                                  SI[s]).wait()
            pltpu.make_async_copy(gin_h.at[pl.ds(eb, CH)], GB[s],
                                  SI[s]).wait()

        def issue_gather(s):
            pltpu.async_copy(stab_h.at[IS[s]], SR[s], SG[s])
            pltpu.async_copy(dtab_h.at[ID[s]], DR[s], SG[s])

        def drain_gather(s):
            pltpu.make_async_copy(stab_h.at[IS[s]], SR[s], SG[s]).wait()
            pltpu.make_async_copy(dtab_h.at[ID[s]], DR[s], SG[s]).wait()

        def issue_gate(c, s):
            eb = base + c * CH
            pltpu.async_copy(GO[s], gout_h.at[pl.ds(eb, CH)], ST[s])

        def drain_gate(c, s):
            eb = base + c * CH
            pltpu.make_async_copy(GO[s], gout_h.at[pl.ds(eb, CH)],
                                  ST[s]).wait()

        def compute(s):
            for g in range(CH // L):
                rows = lax.iota(jnp.int32, L) + g * L
                sc = GB[s][pl.ds(g * L, L)]
                gate = (sc - mn) * gscale
                if store_gate:
                    GO[s][pl.ds(g * L, L)] = gate
                for hh in range(heads):
                    elh = plsc.load_gather(SR[s], [rows, _col(64 + hh)])
                    if heads == 1:
                        erh = plsc.load_gather(DR[s], [rows, _col(0)])
                    else:
                        erh = plsc.load_gather(DR[s], [rows, _col(8 + hh)])
                    e = _lrelu(elh + erh)
                    ex = jnp.exp(e * gate)
                    w = 64 // heads
                    for c in range(hh * w, (hh + 1) * w):
                        zc = plsc.load_gather(SR[s], [rows, _col(c)])
                        plsc.store_scatter(obuf, [rows, _col(c)], ex * zc)
                    plsc.store_scatter(obuf, [rows, _col(64 + hh)], ex)

        def chunk(c, b, first, n1, n2):
            drain_gather(b)
            if n1:
                drain_idx(c + 1, 1 - b)
                issue_gather(1 - b)
            if store_gate and not first:
                drain_gate(c - 2, b)
            compute(b)
            pltpu.sync_copy(obuf, shacc.at[ID[b]], add=True)
            if store_gate:
                issue_gate(c, b)
            if n2 == 'always':
                issue_idx(c + 2, b)
            elif n2 == 'cond':
                @pl.when(c + 2 < NCHUNK)
                def _():
                    issue_idx(c + 2, b)

        issue_idx(0, 0)
        drain_idx(0, 0)
        issue_gather(0)
        issue_idx(1, 1)
        chunk(0, 0, True, True, 'always')
        chunk(1, 1, True, True, 'always')

        def pair(j, _):
            c0 = 2 * j
            chunk(c0, 0, False, True, 'always')
            chunk(c0 + 1, 1, False, True, 'cond')
            return 0

        lax.fori_loop(1, (NCHUNK - 1) // 2, pair, 0)
        chunk(NCHUNK - 1, 0, False, False, 'no')
        if store_gate:
            drain_gate(NCHUNK - 2, 1)
            drain_gate(NCHUNK - 1, 0)

        plsc.subcore_barrier()
        for j in range(NRC):
            r0 = sid * NR + j * 125
            pltpu.sync_copy(shacc.at[pl.ds(r0, 125)],
                            acc_h.at[cid, pl.ds(r0, 125)])

    return scpass


def _make_sc4(E, n):
    EP = E // NW
    NCHUNK = EP // CH

    @functools.partial(
        pl.kernel,
        mesh=_mesh(),
        compiler_params=pltpu.CompilerParams(needs_layout_passes=False, use_tc_tiling_on_sc=False),
        out_type=[_f32(E, 2)],
        scratch_types=[
            pltpu.VMEM((CH,), jnp.int32),
            pltpu.VMEM((CH,), jnp.int32),
            pltpu.VMEM((CH,), jnp.int32),
            pltpu.VMEM((CH,), jnp.int32),
            pltpu.VMEM((CH, 16), jnp.float32),
            pltpu.VMEM((CH, 16), jnp.float32),
            pltpu.VMEM((CH, 16), jnp.float32),
            pltpu.VMEM((CH, 16), jnp.float32),
            pltpu.VMEM((CH, 2), jnp.float32),
            pltpu.VMEM((CH, 2), jnp.float32),
            pltpu.VMEM((16,), jnp.float32),
            pltpu.SemaphoreType.DMA,
            pltpu.SemaphoreType.DMA,
            pltpu.SemaphoreType.DMA,
            pltpu.SemaphoreType.DMA,
            pltpu.SemaphoreType.DMA,
            pltpu.SemaphoreType.DMA,
        ],
    )
    def sc4(src_h, dst_h, ptab_h, qtab_h, prm_h, out_h,
            idx_s0, idx_s1, idx_d0, idx_d1, prows0, prows1, qrows0, qrows1,
            obuf0, obuf1, prm_v,
            sem_i0, sem_i1, sem_g0, sem_g1, sem_t0, sem_t1):
        wid = _wid()
        base = wid * EP
        IS = [idx_s0, idx_s1]
        ID = [idx_d0, idx_d1]
        PR = [prows0, prows1]
        QR = [qrows0, qrows1]
        OB = [obuf0, obuf1]
        SI = [sem_i0, sem_i1]
        SG = [sem_g0, sem_g1]
        ST = [sem_t0, sem_t1]
        pltpu.sync_copy(prm_h, prm_v)
        pv = prm_v[pl.ds(0, L)]
        bp0 = pv[0]
        bp1 = pv[1]

        def issue_idx(c, s):
            eb = base + c * CH
            pltpu.async_copy(src_h.at[pl.ds(eb, CH)], IS[s], SI[s])
            pltpu.async_copy(dst_h.at[pl.ds(eb, CH)], ID[s], SI[s])

        def drain_idx(c, s):
            eb = base + c * CH
            pltpu.make_async_copy(src_h.at[pl.ds(eb, CH)], IS[s],
                                  SI[s]).wait()
            pltpu.make_async_copy(dst_h.at[pl.ds(eb, CH)], ID[s],
                                  SI[s]).wait()

        def issue_gather(s):
            pltpu.async_copy(ptab_h.at[IS[s]], PR[s], SG[s])
            pltpu.async_copy(qtab_h.at[ID[s]], QR[s], SG[s])

        def drain_gather(s):
            pltpu.make_async_copy(ptab_h.at[IS[s]], PR[s], SG[s]).wait()
            pltpu.make_async_copy(qtab_h.at[ID[s]], QR[s], SG[s]).wait()

        def issue_store(c, s):
            eb = base + c * CH
            pltpu.async_copy(OB[s], out_h.at[pl.ds(eb, CH)], ST[s])

        def drain_store(c, s):
            eb = base + c * CH
            pltpu.make_async_copy(OB[s], out_h.at[pl.ds(eb, CH)],
                                  ST[s]).wait()

        def chunk(c, b, first, n1, n2):
            drain_gather(b)
            if n1:
                drain_idx(c + 1, 1 - b)
                issue_gather(1 - b)
            if n2 == 'always':
                issue_idx(c + 2, b)
            elif n2 == 'cond':
                @pl.when(c + 2 < NCHUNK)
                def _():
                    issue_idx(c + 2, b)
            if not first:
                drain_store(c - 2, b)
            for g in range(CH // L):
                rows = lax.iota(jnp.int32, L) + g * L
                for cc, bpc in ((0, bp0), (1, bp1)):
                    pc = plsc.load_gather(PR[b], [rows, _col(cc)])
                    qc = plsc.load_gather(QR[b], [rows, _col(cc)])
                    plsc.store_scatter(OB[b], [rows, _col(cc)],
                                       pc + qc + bpc)
            issue_store(c, b)

        issue_idx(0, 0)
        drain_idx(0, 0)
        issue_gather(0)
        issue_idx(1, 1)
        chunk(0, 0, True, True, 'always')
        chunk(1, 1, True, True, 'always')

        def pair(j, _):
            c0 = 2 * j
            chunk(c0, 0, False, True, 'always')
            chunk(c0 + 1, 1, False, True, 'cond')
            return 0

        lax.fori_loop(1, (NCHUNK - 1) // 2, pair, 0)
        chunk(NCHUNK - 1, 0, False, False, 'no')
        drain_store(NCHUNK - 2, 1)
        drain_store(NCHUNK - 1, 0)

    return sc4


def kernel(h, edge_index, W1, b1, W2, b2, W3, b3, fc1, attn1, fc2, attn2,
           Wp, bp):
    n, d = h.shape
    E = edge_index.shape[1]
    nh, _, hd = fc1.shape
    src = edge_index[0]
    dst = edge_index[1]

    f1 = jnp.transpose(fc1, (1, 0, 2)).reshape(d, nh * hd)
    eye = jnp.eye(nh, dtype=jnp.float32)
    al = (attn1[:, :hd, 0][:, :, None] * eye[:, None, :]).reshape(nh * hd, nh)
    ar = (attn1[:, hd:, 0][:, :, None] * eye[:, None, :]).reshape(nh * hd, nh)
    w2a = W2[:16]
    w2b = W2[16:]

    u, v, z_all, el, er = pl.pallas_call(
        _tc0,
        out_shape=[_f32(n, 8), _f32(n, 8), _f32(n, 64), _f32(n, 4),
                   _f32(n, 4)],
    )(h, W1, b1, w2a, w2b, f1, al, ar)

    zpad4 = jnp.zeros((n, 4), jnp.float32)
    zpad8 = jnp.zeros((n, 8), jnp.float32)
    zpad12 = jnp.zeros((n, 12), jnp.float32)
    utab = jnp.concatenate([u, zpad8], axis=1)
    dtab = jnp.concatenate([v, er, zpad4], axis=1)
    s1tab = jnp.concatenate([z_all, el, zpad12], axis=1)

    prm1 = jnp.concatenate([b2, W3[:, 0], b3,
                            jnp.zeros((15,), jnp.float32)])

    score, mm = _make_sc1(E, n)(src, dst, utab, dtab, prm1)
    mn = jnp.min(mm[:, 0])
    mx = jnp.max(mm[:, 1])
    gscale = 1.0 / (mx - mn)
    prm2 = jnp.zeros((16,), jnp.float32).at[0].set(mn).at[1].set(gscale)

    gate, acc1 = _make_gat_pass(E, n, nh, True)(src, dst, s1tab, dtab,
                                               score, prm2)

    z2, el2, er2 = pl.pallas_call(
        _tc1,
        out_shape=[_f32(n, 64), _f32(n, 1), _f32(n, 1)],
    )(acc1, fc2, attn2[:64], attn2[64:])

    s2tab = jnp.concatenate([z2, el2, jnp.zeros((n, 15), jnp.float32)],
                            axis=1)
    d2tab = jnp.concatenate([er2, jnp.zeros((n, 15), jnp.float32)], axis=1)

    _, acc2 = _make_gat_pass(E, n, 1, False)(src, dst, s2tab, d2tab,
                                             score, prm2)

    p, q = pl.pallas_call(
        _tc2,
        out_shape=[_f32(n, 2), _f32(n, 2)],
    )(acc2, Wp[:64], Wp[64:])

    ptab = jnp.concatenate([p, jnp.zeros((n, 14), jnp.float32)], axis=1)
    qtab = jnp.concatenate([q, jnp.zeros((n, 14), jnp.float32)], axis=1)
    prm4 = jnp.zeros((16,), jnp.float32).at[0].set(bp[0]).at[1].set(bp[1])

    (escore,) = _make_sc4(E, n)(src, dst, ptab, qtab, prm4)

    return escore, gate[:, None]


# speedup vs baseline: 1.1068x; 1.1068x over previous
import functools
import jax
import jax.numpy as jnp
from jax import lax
from jax.experimental import pallas as pl
from jax.experimental.pallas import tpu as pltpu
from jax.experimental.pallas import tpu_sc as plsc

NC = 2
NS = 16
NW = NC * NS
L = 16
CH = 80


def _f32(*shape):
    return jax.ShapeDtypeStruct(shape, jnp.float32)


def _mesh():
    return plsc.VectorSubcoreMesh(core_axis_name="c", subcore_axis_name="s")


def _wid():
    return lax.axis_index("s") * NC + lax.axis_index("c")


def _col(c):
    return jnp.full((L,), c, jnp.int32)


def _lrelu(x):
    return jnp.where(x > 0, x, 0.01 * x)


def _tc0(h, w1, b1, w2a, w2b, f1, al, ar, utab_ref, dtab_ref, s1tab_ref):
    hv = h[...]
    nn = hv.shape[0]
    z8 = jnp.zeros((nn, 8), jnp.float32)
    z4 = jnp.zeros((nn, 4), jnp.float32)
    z12 = jnp.zeros((nn, 12), jnp.float32)
    hg = jnp.dot(hv, w1[...], preferred_element_type=jnp.float32) + b1[...]
    u = jnp.dot(hg, w2a[...], preferred_element_type=jnp.float32)
    v = jnp.dot(hg, w2b[...], preferred_element_type=jnp.float32)
    z = jnp.dot(hv, f1[...], preferred_element_type=jnp.float32)
    el = jnp.dot(z, al[...], preferred_element_type=jnp.float32)
    er = jnp.dot(z, ar[...], preferred_element_type=jnp.float32)
    utab_ref[...] = jnp.concatenate([u, z8], axis=1)
    dtab_ref[...] = jnp.concatenate([v, er, z4], axis=1)
    s1tab_ref[...] = jnp.concatenate([z, el, z12], axis=1)


def _tc1(acc, fc2, a2l, a2r, s2tab_ref, d2tab_ref):
    a = acc[0] + acc[1]
    num = a[:, :64]
    den = a[:, 64:68]
    den = jnp.where(den == 0.0, 1.0, den)
    n = num.shape[0]
    den_rep = jnp.concatenate(
        [jnp.broadcast_to(den[:, i:i + 1], (n, 16)) for i in range(4)], axis=1)
    h1 = _lrelu(num / den_rep)
    z15 = jnp.zeros((n, 15), jnp.float32)
    z2 = jnp.dot(h1, fc2[...], preferred_element_type=jnp.float32)
    el2 = jnp.dot(z2, a2l[...], preferred_element_type=jnp.float32)
    er2 = jnp.dot(z2, a2r[...], preferred_element_type=jnp.float32)
    s2tab_ref[...] = jnp.concatenate([z2, el2, z15], axis=1)
    d2tab_ref[...] = jnp.concatenate([er2, z15], axis=1)


def _tc2(acc, wpl, wpr, ptab_ref, qtab_ref):
    a = acc[0] + acc[1]
    den = a[:, 64:65]
    den = jnp.where(den == 0.0, 1.0, den)
    h2 = a[:, :64] / den
    n = h2.shape[0]
    z14 = jnp.zeros((n, 14), jnp.float32)
    p = jnp.dot(h2, wpl[...], preferred_element_type=jnp.float32)
    q = jnp.dot(h2, wpr[...], preferred_element_type=jnp.float32)
    ptab_ref[...] = jnp.concatenate([p, z14], axis=1)
    qtab_ref[...] = jnp.concatenate([q, z14], axis=1)


def _zero_lane16(buf, rows, c0):
    z = jnp.zeros((L,), jnp.float32)

    def body(r, _):
        buf[r, pl.ds(c0, L)] = z
        return 0

    lax.fori_loop(0, rows, body, 0)


def _make_sc1(E, n):
    EP = E // NW
    NCHUNK = EP // CH

    @functools.partial(
        pl.kernel,
        mesh=_mesh(),
        compiler_params=pltpu.CompilerParams(needs_layout_passes=False, use_tc_tiling_on_sc=False),
        out_type=[_f32(E), _f32(NW, L)],
        scratch_types=[
            pltpu.VMEM((CH,), jnp.int32),
            pltpu.VMEM((CH,), jnp.int32),
            pltpu.VMEM((CH,), jnp.int32),
            pltpu.VMEM((CH,), jnp.int32),
            pltpu.VMEM((CH, 16), jnp.float32),
            pltpu.VMEM((CH, 16), jnp.float32),
            pltpu.VMEM((CH, 16), jnp.float32),
            pltpu.VMEM((CH, 16), jnp.float32),
            pltpu.VMEM((CH,), jnp.float32),
            pltpu.VMEM((CH,), jnp.float32),
            pltpu.VMEM((32,), jnp.float32),
            pltpu.VMEM((L,), jnp.float32),
            pltpu.SemaphoreType.DMA,
            pltpu.SemaphoreType.DMA,
            pltpu.SemaphoreType.DMA,
            pltpu.SemaphoreType.DMA,
            pltpu.SemaphoreType.DMA,
            pltpu.SemaphoreType.DMA,
        ],
    )
    def sc1(src_h, dst_h, utab_h, dtab_h, prm_h, score_h, mm_h,
            idx_s0, idx_s1, idx_d0, idx_d1, urows0, urows1, drows0, drows1,
            sbuf0, sbuf1, prm_v, mmbuf,
            sem_i0, sem_i1, sem_g0, sem_g1, sem_t0, sem_t1):
        wid = _wid()
        base = wid * EP
        IS = [idx_s0, idx_s1]
        ID = [idx_d0, idx_d1]
        UR = [urows0, urows1]
        DR = [drows0, drows1]
        SB = [sbuf0, sbuf1]
        SI = [sem_i0, sem_i1]
        SG = [sem_g0, sem_g1]
        ST = [sem_t0, sem_t1]
        pltpu.sync_copy(prm_h, prm_v)
        pva = prm_v[pl.ds(0, L)]
        pvb = prm_v[pl.ds(L, L)]
        b2 = [pva[k] for k in range(8)]
        w3 = [pva[8 + k] for k in range(8)]
        b3 = pvb[0]

        def issue_idx(c, s):
            eb = base + c * CH
            pltpu.async_copy(src_h.at[pl.ds(eb, CH)], IS[s], SI[s])
            pltpu.async_copy(dst_h.at[pl.ds(eb, CH)], ID[s], SI[s])

        def drain_idx(c, s):
            eb = base + c * CH
            pltpu.make_async_copy(src_h.at[pl.ds(eb, CH)], IS[s],
                                  SI[s]).wait()
            pltpu.make_async_copy(dst_h.at[pl.ds(eb, CH)], ID[s],
                                  SI[s]).wait()

        def issue_gather(s):
            pltpu.async_copy(utab_h.at[IS[s]], UR[s], SG[s])
            pltpu.async_copy(dtab_h.at[ID[s]], DR[s], SG[s])

        def drain_gather(s):
            pltpu.make_async_copy(utab_h.at[IS[s]], UR[s], SG[s]).wait()
            pltpu.make_async_copy(dtab_h.at[ID[s]], DR[s], SG[s]).wait()

        def issue_store(c, s):
            eb = base + c * CH
            pltpu.async_copy(SB[s], score_h.at[pl.ds(eb, CH)], ST[s])

        def drain_store(c, s):
            eb = base + c * CH
            pltpu.make_async_copy(SB[s], score_h.at[pl.ds(eb, CH)],
                                  ST[s]).wait()

        def chunk(c, b, first, n1, n2, carry):
            vmin, vmax = carry
            drain_gather(b)
            if n1:
                drain_idx(c + 1, 1 - b)
                issue_gather(1 - b)
            if n2 == 'always':
                issue_idx(c + 2, b)
            elif n2 == 'cond':
                @pl.when(c + 2 < NCHUNK)
                def _():
                    issue_idx(c + 2, b)
            if not first:
                drain_store(c - 2, b)
            for g in range(CH // L):
                rows = lax.iota(jnp.int32, L) + g * L
                acc = jnp.full((L,), 0.0, jnp.float32) + b3
                for k in range(8):
                    uk = plsc.load_gather(UR[b], [rows, _col(k)])
                    vk = plsc.load_gather(DR[b], [rows, _col(k)])
                    s = jnp.maximum(uk + vk + b2[k], 0.0)
                    acc = acc + s * w3[k]
                SB[b][pl.ds(g * L, L)] = acc
                vmin = jnp.minimum(vmin, acc)
                vmax = jnp.maximum(vmax, acc)
            issue_store(c, b)
            return vmin, vmax

        init = (jnp.full((L,), jnp.inf, jnp.float32),
                jnp.full((L,), -jnp.inf, jnp.float32))
        issue_idx(0, 0)
        drain_idx(0, 0)
        issue_gather(0)
        issue_idx(1, 1)
        carry = chunk(0, 0, True, True, 'always', init)
        carry = chunk(1, 1, True, True, 'always', carry)

        def pair(j, carry):
            c0 = 2 * j
            carry = chunk(c0, 0, False, True, 'always', carry)
            carry = chunk(c0 + 1, 1, False, True, 'cond', carry)
            return carry

        carry = lax.fori_loop(1, (NCHUNK - 1) // 2, pair, carry)
        vmin, vmax = chunk(NCHUNK - 1, 0, False, False, 'no', carry)
        drain_store(NCHUNK - 2, 1)
        drain_store(NCHUNK - 1, 0)
        mn = jnp.min(vmin)
        mx = jnp.max(vmax)
        lane = lax.iota(jnp.int32, L)
        mmbuf[...] = jnp.where(lane == 0, mn, jnp.where(lane == 1, mx, 0.0))
        pltpu.sync_copy(mmbuf, mm_h.at[wid])

    return sc1


def _make_gat_pass(E, n, heads, store_gate):
    EP = E // NW
    NCHUNK = EP // CH
    NR = n // NS
    NRC = NR // 125

    @functools.partial(
        pl.kernel,
        mesh=_mesh(),
        compiler_params=pltpu.CompilerParams(needs_layout_passes=False, use_tc_tiling_on_sc=False),
        out_type=[_f32(E), _f32(NC, n, 80)],
        scratch_types=[
            pltpu.VMEM((CH,), jnp.int32),
            pltpu.VMEM((CH,), jnp.int32),
            pltpu.VMEM((CH,), jnp.int32),
            pltpu.VMEM((CH,), jnp.int32),
            pltpu.VMEM((CH, 80), jnp.float32),
            pltpu.VMEM((CH, 80), jnp.float32),
            pltpu.VMEM((CH, 16), jnp.float32),
            pltpu.VMEM((CH, 16), jnp.float32),
            pltpu.VMEM((CH,), jnp.float32),
            pltpu.VMEM((CH,), jnp.float32),
            pltpu.VMEM((CH,), jnp.float32),
            pltpu.VMEM((CH,), jnp.float32),
            pltpu.VMEM((CH, 80), jnp.float32),
            pltpu.VMEM((16,), jnp.float32),
            pltpu.VMEM((125, 80), jnp.float32),
            pltpu.VMEM_SHARED((n, 80), jnp.float32),
            pltpu.SemaphoreType.DMA,
            pltpu.SemaphoreType.DMA,
            pltpu.SemaphoreType.DMA,
            pltpu.SemaphoreType.DMA,
            pltpu.SemaphoreType.DMA,
            pltpu.SemaphoreType.DMA,
        ],
    )
    def scpass(src_h, dst_h, stab_h, dtab_h, gin_h, prm_h, gout_h, acc_h,
               idx_s0, idx_s1, idx_d0, idx_d1, srows0, srows1, drows0,
               drows1, gbuf0, gbuf1, gobuf0, gobuf1, obuf, prm_v, zb,
               shacc,
               sem_i0, sem_i1, sem_g0, sem_g1, sem_t0, sem_t1):
        cid = lax.axis_index("c")
        sid = lax.axis_index("s")
        wid = sid * NC + cid
        base = wid * EP
        IS = [idx_s0, idx_s1]
        ID = [idx_d0, idx_d1]
        SR = [srows0, srows1]
        DR = [drows0, drows1]
        GB = [gbuf0, gbuf1]
        GO = [gobuf0, gobuf1]
        SI = [sem_i0, sem_i1]
        SG = [sem_g0, sem_g1]
        ST = [sem_t0, sem_t1]

        pltpu.sync_copy(prm_h, prm_v)
        pv = prm_v[pl.ds(0, L)]
        mn = pv[0]
        gscale = pv[1]

        for c5 in range(5):
            _zero_lane16(zb, 125, c5 * L)
        for j in range(NRC):
            pltpu.sync_copy(zb, shacc.at[pl.ds(sid * NR + j * 125, 125)])
        _zero_lane16(obuf, CH, 64)
        plsc.subcore_barrier()

        def issue_idx(c, s):
            eb = base + c * CH
            pltpu.async_copy(src_h.at[pl.ds(eb, CH)], IS[s], SI[s])
            pltpu.async_copy(dst_h.at[pl.ds(eb, CH)], ID[s], SI[s])
            pltpu.async_copy(gin_h.at[pl.ds(eb, CH)], GB[s], SI[s])

        def drain_idx(c, s):
            eb = base + c * CH
            pltpu.make_async_copy(src_h.at[pl.ds(eb, CH)], IS[s],
                                  SI[s]).wait()
            pltpu.make_async_copy(dst_h.at[pl.ds(eb, CH)], ID[s],
                                  SI[s]).wait()
            pltpu.make_async_copy(gin_h.at[pl.ds(eb, CH)], GB[s],
                                  SI[s]).wait()

        def issue_gather(s):
            pltpu.async_copy(stab_h.at[IS[s]], SR[s], SG[s])
            pltpu.async_copy(dtab_h.at[ID[s]], DR[s], SG[s])

        def drain_gather(s):
            pltpu.make_async_copy(stab_h.at[IS[s]], SR[s], SG[s]).wait()
            pltpu.make_async_copy(dtab_h.at[ID[s]], DR[s], SG[s]).wait()

        def issue_gate(c, s):
            eb = base + c * CH
            pltpu.async_copy(GO[s], gout_h.at[pl.ds(eb, CH)], ST[s])

        def drain_gate(c, s):
            eb = base + c * CH
            pltpu.make_async_copy(GO[s], gout_h.at[pl.ds(eb, CH)],
                                  ST[s]).wait()

        def compute(s):
            def grp(g, _):
                rows = lax.iota(jnp.int32, L) + g * L
                sc = GB[s][pl.ds(g * L, L)]
                gate = (sc - mn) * gscale
                if store_gate:
                    GO[s][pl.ds(g * L, L)] = gate
                for hh in range(heads):
                    elh = plsc.load_gather(SR[s], [rows, _col(64 + hh)])
                    if heads == 1:
                        erh = plsc.load_gather(DR[s], [rows, _col(0)])
                    else:
                        erh = plsc.load_gather(DR[s], [rows, _col(8 + hh)])
                    e = _lrelu(elh + erh)
                    ex = jnp.exp(e * gate)
                    w = 64 // heads
                    for c in range(hh * w, (hh + 1) * w):
                        zc = plsc.load_gather(SR[s], [rows, _col(c)])
                        plsc.store_scatter(obuf, [rows, _col(c)], ex * zc)
                    plsc.store_scatter(obuf, [rows, _col(64 + hh)], ex)
                return 0

            lax.fori_loop(0, CH // L, grp, 0)

        def chunk(c, b, first, n1, n2):
            drain_gather(b)
            if n1:
                drain_idx(c + 1, 1 - b)
                issue_gather(1 - b)
            if store_gate and not first:
                drain_gate(c - 2, b)
            compute(b)
            pltpu.sync_copy(obuf, shacc.at[ID[b]], add=True)
            if store_gate:
                issue_gate(c, b)
            if n2 == 'always':
                issue_idx(c + 2, b)
            elif n2 == 'cond':
                @pl.when(c + 2 < NCHUNK)
                def _():
                    issue_idx(c + 2, b)

        issue_idx(0, 0)
        drain_idx(0, 0)
        issue_gather(0)
        issue_idx(1, 1)
        chunk(0, 0, True, True, 'always')
        chunk(1, 1, True, True, 'always')

        def pair(j, _):
            c0 = 2 * j
            chunk(c0, 0, False, True, 'always')
            chunk(c0 + 1, 1, False, True, 'cond')
            return 0

        lax.fori_loop(1, (NCHUNK - 1) // 2, pair, 0)
        chunk(NCHUNK - 1, 0, False, False, 'no')
        if store_gate:
            drain_gate(NCHUNK - 2, 1)
            drain_gate(NCHUNK - 1, 0)

        plsc.subcore_barrier()
        for j in range(NRC):
            r0 = sid * NR + j * 125
            pltpu.sync_copy(shacc.at[pl.ds(r0, 125)],
                            acc_h.at[cid, pl.ds(r0, 125)])

    return scpass


def _make_sc4(E, n):
    EP = E // NW
    NCHUNK = EP // CH

    @functools.partial(
        pl.kernel,
        mesh=_mesh(),
        compiler_params=pltpu.CompilerParams(needs_layout_passes=False, use_tc_tiling_on_sc=False),
        out_type=[_f32(E, 2)],
        scratch_types=[
            pltpu.VMEM((CH,), jnp.int32),
            pltpu.VMEM((CH,), jnp.int32),
            pltpu.VMEM((CH,), jnp.int32),
            pltpu.VMEM((CH,), jnp.int32),
            pltpu.VMEM((CH, 16), jnp.float32),
            pltpu.VMEM((CH, 16), jnp.float32),
            pltpu.VMEM((CH, 16), jnp.float32),
            pltpu.VMEM((CH, 16), jnp.float32),
            pltpu.VMEM((CH, 2), jnp.float32),
            pltpu.VMEM((CH, 2), jnp.float32),
            pltpu.VMEM((16,), jnp.float32),
            pltpu.SemaphoreType.DMA,
            pltpu.SemaphoreType.DMA,
            pltpu.SemaphoreType.DMA,
            pltpu.SemaphoreType.DMA,
            pltpu.SemaphoreType.DMA,
            pltpu.SemaphoreType.DMA,
        ],
    )
    def sc4(src_h, dst_h, ptab_h, qtab_h, prm_h, out_h,
            idx_s0, idx_s1, idx_d0, idx_d1, prows0, prows1, qrows0, qrows1,
            obuf0, obuf1, prm_v,
            sem_i0, sem_i1, sem_g0, sem_g1, sem_t0, sem_t1):
        wid = _wid()
        base = wid * EP
        IS = [idx_s0, idx_s1]
        ID = [idx_d0, idx_d1]
        PR = [prows0, prows1]
        QR = [qrows0, qrows1]
        OB = [obuf0, obuf1]
        SI = [sem_i0, sem_i1]
        SG = [sem_g0, sem_g1]
        ST = [sem_t0, sem_t1]
        pltpu.sync_copy(prm_h, prm_v)
        pv = prm_v[pl.ds(0, L)]
        bp0 = pv[0]
        bp1 = pv[1]

        def issue_idx(c, s):
            eb = base + c * CH
            pltpu.async_copy(src_h.at[pl.ds(eb, CH)], IS[s], SI[s])
            pltpu.async_copy(dst_h.at[pl.ds(eb, CH)], ID[s], SI[s])

        def drain_idx(c, s):
            eb = base + c * CH
            pltpu.make_async_copy(src_h.at[pl.ds(eb, CH)], IS[s],
                                  SI[s]).wait()
            pltpu.make_async_copy(dst_h.at[pl.ds(eb, CH)], ID[s],
                                  SI[s]).wait()

        def issue_gather(s):
            pltpu.async_copy(ptab_h.at[IS[s]], PR[s], SG[s])
            pltpu.async_copy(qtab_h.at[ID[s]], QR[s], SG[s])

        def drain_gather(s):
            pltpu.make_async_copy(ptab_h.at[IS[s]], PR[s], SG[s]).wait()
            pltpu.make_async_copy(qtab_h.at[ID[s]], QR[s], SG[s]).wait()

        def issue_store(c, s):
            eb = base + c * CH
            pltpu.async_copy(OB[s], out_h.at[pl.ds(eb, CH)], ST[s])

        def drain_store(c, s):
            eb = base + c * CH
            pltpu.make_async_copy(OB[s], out_h.at[pl.ds(eb, CH)],
                                  ST[s]).wait()

        def chunk(c, b, first, n1, n2):
            drain_gather(b)
            if n1:
                drain_idx(c + 1, 1 - b)
                issue_gather(1 - b)
            if n2 == 'always':
                issue_idx(c + 2, b)
            elif n2 == 'cond':
                @pl.when(c + 2 < NCHUNK)
                def _():
                    issue_idx(c + 2, b)
            if not first:
                drain_store(c - 2, b)
            for g in range(CH // L):
                rows = lax.iota(jnp.int32, L) + g * L
                for cc, bpc in ((0, bp0), (1, bp1)):
                    pc = plsc.load_gather(PR[b], [rows, _col(cc)])
                    qc = plsc.load_gather(QR[b], [rows, _col(cc)])
                    plsc.store_scatter(OB[b], [rows, _col(cc)],
                                       pc + qc + bpc)
            issue_store(c, b)

        issue_idx(0, 0)
        drain_idx(0, 0)
        issue_gather(0)
        issue_idx(1, 1)
        chunk(0, 0, True, True, 'always')
        chunk(1, 1, True, True, 'always')

        def pair(j, _):
            c0 = 2 * j
            chunk(c0, 0, False, True, 'always')
            chunk(c0 + 1, 1, False, True, 'cond')
            return 0

        lax.fori_loop(1, (NCHUNK - 1) // 2, pair, 0)
        chunk(NCHUNK - 1, 0, False, False, 'no')
        drain_store(NCHUNK - 2, 1)
        drain_store(NCHUNK - 1, 0)

    return sc4


def kernel(h, edge_index, W1, b1, W2, b2, W3, b3, fc1, attn1, fc2, attn2,
           Wp, bp):
    n, d = h.shape
    E = edge_index.shape[1]
    nh, _, hd = fc1.shape
    src = edge_index[0]
    dst = edge_index[1]

    f1 = jnp.transpose(fc1, (1, 0, 2)).reshape(d, nh * hd)
    eye = jnp.eye(nh, dtype=jnp.float32)
    al = (attn1[:, :hd, 0][:, :, None] * eye[:, None, :]).reshape(nh * hd, nh)
    ar = (attn1[:, hd:, 0][:, :, None] * eye[:, None, :]).reshape(nh * hd, nh)
    w2a = W2[:16]
    w2b = W2[16:]

    utab, dtab, s1tab = pl.pallas_call(
        _tc0,
        out_shape=[_f32(n, 16), _f32(n, 16), _f32(n, 80)],
    )(h, W1, b1, w2a, w2b, f1, al, ar)

    prm1 = jnp.concatenate([b2, W3[:, 0], b3,
                            jnp.zeros((15,), jnp.float32)])

    score, mm = _make_sc1(E, n)(src, dst, utab, dtab, prm1)
    mn = jnp.min(mm[:, 0])
    mx = jnp.max(mm[:, 1])
    gscale = 1.0 / (mx - mn)
    prm2 = jnp.zeros((16,), jnp.float32).at[0].set(mn).at[1].set(gscale)

    gate, acc1 = _make_gat_pass(E, n, nh, True)(src, dst, s1tab, dtab,
                                               score, prm2)

    s2tab, d2tab = pl.pallas_call(
        _tc1,
        out_shape=[_f32(n, 80), _f32(n, 16)],
    )(acc1, fc2, attn2[:64], attn2[64:])

    _, acc2 = _make_gat_pass(E, n, 1, False)(src, dst, s2tab, d2tab,
                                             score, prm2)

    ptab, qtab = pl.pallas_call(
        _tc2,
        out_shape=[_f32(n, 16), _f32(n, 16)],
    )(acc2, Wp[:64], Wp[64:])
    prm4 = jnp.zeros((16,), jnp.float32).at[0].set(bp[0]).at[1].set(bp[1])

    (escore,) = _make_sc4(E, n)(src, dst, ptab, qtab, prm4)

    return escore, gate[:, None]


# speedup vs baseline: 1.3569x; 1.2259x over previous
import functools
import jax
import jax.numpy as jnp
from jax import lax
from jax.experimental import pallas as pl
from jax.experimental.pallas import tpu as pltpu
from jax.experimental.pallas import tpu_sc as plsc

NC = 2
NS = 16
NW = NC * NS
L = 16
CH = 80


def _f32(*shape):
    return jax.ShapeDtypeStruct(shape, jnp.float32)


def _mesh():
    return plsc.VectorSubcoreMesh(core_axis_name="c", subcore_axis_name="s")


def _wid():
    return lax.axis_index("s") * NC + lax.axis_index("c")


def _col(c):
    return jnp.full((L,), c, jnp.int32)


def _lrelu(x):
    return jnp.where(x > 0, x, 0.01 * x)


def _tc0(h, w1, b1, w2a, w2b, f1, al, ar, utab_ref, dtab_ref, el_ref,
         zbf_ref):
    hv = h[...]
    nn = hv.shape[0]
    z8 = jnp.zeros((nn, 8), jnp.float32)
    z4 = jnp.zeros((nn, 4), jnp.float32)
    hg = jnp.dot(hv, w1[...], preferred_element_type=jnp.float32) + b1[...]
    u = jnp.dot(hg, w2a[...], preferred_element_type=jnp.float32)
    v = jnp.dot(hg, w2b[...], preferred_element_type=jnp.float32)
    z = jnp.dot(hv, f1[...], preferred_element_type=jnp.float32)
    el = jnp.dot(z, al[...], preferred_element_type=jnp.float32)
    er = jnp.dot(z, ar[...], preferred_element_type=jnp.float32)
    utab_ref[...] = jnp.concatenate([u, z8], axis=1)
    dtab_ref[...] = jnp.concatenate([v, er, z4], axis=1)
    el_ref[...] = el
    zbf_ref[...] = z.astype(jnp.bfloat16)


def _tc1(acc, fc2, a2l, a2r, el2_ref, z2bf_ref, d2tab_ref):
    a = acc[0] + acc[1]
    num = a[:, :64]
    den = a[:, 64:68]
    den = jnp.where(den == 0.0, 1.0, den)
    n = num.shape[0]
    den_rep = jnp.concatenate(
        [jnp.broadcast_to(den[:, i:i + 1], (n, 16)) for i in range(4)], axis=1)
    h1 = _lrelu(num / den_rep)
    z15 = jnp.zeros((n, 15), jnp.float32)
    z2 = jnp.dot(h1, fc2[...], preferred_element_type=jnp.float32)
    el2 = jnp.dot(z2, a2l[...], preferred_element_type=jnp.float32)
    er2 = jnp.dot(z2, a2r[...], preferred_element_type=jnp.float32)
    el2_ref[...] = el2
    z2bf_ref[...] = z2.astype(jnp.bfloat16)
    d2tab_ref[...] = jnp.concatenate([er2, z15], axis=1)


def _tc2(acc, wpl, wpr, ptab_ref, qtab_ref):
    a = acc[0] + acc[1]
    den = a[:, 64:65]
    den = jnp.where(den == 0.0, 1.0, den)
    h2 = a[:, :64] / den
    n = h2.shape[0]
    z14 = jnp.zeros((n, 14), jnp.float32)
    p = jnp.dot(h2, wpl[...], preferred_element_type=jnp.float32)
    q = jnp.dot(h2, wpr[...], preferred_element_type=jnp.float32)
    ptab_ref[...] = jnp.concatenate([p, z14], axis=1)
    qtab_ref[...] = jnp.concatenate([q, z14], axis=1)


def _zero_lane16(buf, rows, c0):
    z = jnp.zeros((L,), jnp.float32)

    def body(r, _):
        buf[r, pl.ds(c0, L)] = z
        return 0

    lax.fori_loop(0, rows, body, 0)


def _make_sc1(E, n):
    EP = E // NW
    NCHUNK = EP // CH

    @functools.partial(
        pl.kernel,
        mesh=_mesh(),
        compiler_params=pltpu.CompilerParams(needs_layout_passes=False, use_tc_tiling_on_sc=False),
        out_type=[_f32(E), _f32(NW, L)],
        scratch_types=[
            pltpu.VMEM((CH,), jnp.int32),
            pltpu.VMEM((CH,), jnp.int32),
            pltpu.VMEM((CH,), jnp.int32),
            pltpu.VMEM((CH,), jnp.int32),
            pltpu.VMEM((CH, 16), jnp.float32),
            pltpu.VMEM((CH, 16), jnp.float32),
            pltpu.VMEM((CH, 16), jnp.float32),
            pltpu.VMEM((CH, 16), jnp.float32),
            pltpu.VMEM((CH,), jnp.float32),
            pltpu.VMEM((CH,), jnp.float32),
            pltpu.VMEM((32,), jnp.float32),
            pltpu.VMEM((L,), jnp.float32),
            pltpu.SemaphoreType.DMA,
            pltpu.SemaphoreType.DMA,
            pltpu.SemaphoreType.DMA,
            pltpu.SemaphoreType.DMA,
            pltpu.SemaphoreType.DMA,
            pltpu.SemaphoreType.DMA,
        ],
    )
    def sc1(src_h, dst_h, utab_h, dtab_h, prm_h, score_h, mm_h,
            idx_s0, idx_s1, idx_d0, idx_d1, urows0, urows1, drows0, drows1,
            sbuf0, sbuf1, prm_v, mmbuf,
            sem_i0, sem_i1, sem_g0, sem_g1, sem_t0, sem_t1):
        wid = _wid()
        base = wid * EP
        IS = [idx_s0, idx_s1]
        ID = [idx_d0, idx_d1]
        UR = [urows0, urows1]
        DR = [drows0, drows1]
        SB = [sbuf0, sbuf1]
        SI = [sem_i0, sem_i1]
        SG = [sem_g0, sem_g1]
        ST = [sem_t0, sem_t1]
        pltpu.sync_copy(prm_h, prm_v)
        pva = prm_v[pl.ds(0, L)]
        pvb = prm_v[pl.ds(L, L)]
        b2 = [pva[k] for k in range(8)]
        w3 = [pva[8 + k] for k in range(8)]
        b3 = pvb[0]

        def issue_idx(c, s):
            eb = base + c * CH
            pltpu.async_copy(src_h.at[pl.ds(eb, CH)], IS[s], SI[s])
            pltpu.async_copy(dst_h.at[pl.ds(eb, CH)], ID[s], SI[s])

        def drain_idx(c, s):
            eb = base + c * CH
            pltpu.make_async_copy(src_h.at[pl.ds(eb, CH)], IS[s],
                                  SI[s]).wait()
            pltpu.make_async_copy(dst_h.at[pl.ds(eb, CH)], ID[s],
                                  SI[s]).wait()

        def issue_gather(s):
            pltpu.async_copy(utab_h.at[IS[s]], UR[s], SG[s])
            pltpu.async_copy(dtab_h.at[ID[s]], DR[s], SG[s])

        def drain_gather(s):
            pltpu.make_async_copy(utab_h.at[IS[s]], UR[s], SG[s]).wait()
            pltpu.make_async_copy(dtab_h.at[ID[s]], DR[s], SG[s]).wait()

        def issue_store(c, s):
            eb = base + c * CH
            pltpu.async_copy(SB[s], score_h.at[pl.ds(eb, CH)], ST[s])

        def drain_store(c, s):
            eb = base + c * CH
            pltpu.make_async_copy(SB[s], score_h.at[pl.ds(eb, CH)],
                                  ST[s]).wait()

        def chunk(c, b, first, n1, n2, carry):
            vmin, vmax = carry
            drain_gather(b)
            if n1:
                drain_idx(c + 1, 1 - b)
                issue_gather(1 - b)
            if n2 == 'always':
                issue_idx(c + 2, b)
            elif n2 == 'cond':
                @pl.when(c + 2 < NCHUNK)
                def _():
                    issue_idx(c + 2, b)
            if not first:
                drain_store(c - 2, b)
            for g in range(CH // L):
                rows = lax.iota(jnp.int32, L) + g * L
                acc = jnp.full((L,), 0.0, jnp.float32) + b3
                for k in range(8):
                    uk = plsc.load_gather(UR[b], [rows, _col(k)])
                    vk = plsc.load_gather(DR[b], [rows, _col(k)])
                    s = jnp.maximum(uk + vk + b2[k], 0.0)
                    acc = acc + s * w3[k]
                SB[b][pl.ds(g * L, L)] = acc
                vmin = jnp.minimum(vmin, acc)
                vmax = jnp.maximum(vmax, acc)
            issue_store(c, b)
            return vmin, vmax

        init = (jnp.full((L,), jnp.inf, jnp.float32),
                jnp.full((L,), -jnp.inf, jnp.float32))
        issue_idx(0, 0)
        drain_idx(0, 0)
        issue_gather(0)
        issue_idx(1, 1)
        carry = chunk(0, 0, True, True, 'always', init)
        carry = chunk(1, 1, True, True, 'always', carry)

        def pair(j, carry):
            c0 = 2 * j
            carry = chunk(c0, 0, False, True, 'always', carry)
            carry = chunk(c0 + 1, 1, False, True, 'cond', carry)
            return carry

        carry = lax.fori_loop(1, (NCHUNK - 1) // 2, pair, carry)
        vmin, vmax = chunk(NCHUNK - 1, 0, False, False, 'no', carry)
        drain_store(NCHUNK - 2, 1)
        drain_store(NCHUNK - 1, 0)
        mn = jnp.min(vmin)
        mx = jnp.max(vmax)
        lane = lax.iota(jnp.int32, L)
        mmbuf[...] = jnp.where(lane == 0, mn, jnp.where(lane == 1, mx, 0.0))
        pltpu.sync_copy(mmbuf, mm_h.at[wid])

    return sc1


def _make_gat_pass(E, n, heads, store_gate):
    EP = E // NW
    NCHUNK = EP // CH
    NR = n // NS
    NRC = NR // 125

    @functools.partial(
        pl.kernel,
        mesh=_mesh(),
        compiler_params=pltpu.CompilerParams(needs_layout_passes=False, use_tc_tiling_on_sc=False),
        out_type=[_f32(E), _f32(NC, n, 80)],
        scratch_types=[
            pltpu.VMEM((CH,), jnp.int32),
            pltpu.VMEM((CH,), jnp.int32),
            pltpu.VMEM((CH,), jnp.int32),
            pltpu.VMEM((CH,), jnp.int32),
            pltpu.VMEM((CH, 48), jnp.float32),
            pltpu.VMEM((CH, 48), jnp.float32),
            pltpu.VMEM((CH, 16), jnp.float32),
            pltpu.VMEM((CH, 16), jnp.float32),
            pltpu.VMEM((CH,), jnp.float32),
            pltpu.VMEM((CH,), jnp.float32),
            pltpu.VMEM((CH,), jnp.float32),
            pltpu.VMEM((CH,), jnp.float32),
            pltpu.VMEM((CH, 80), jnp.float32),
            pltpu.VMEM((16,), jnp.float32),
            pltpu.VMEM((125, 80), jnp.float32),
            pltpu.VMEM_SHARED((n, 80), jnp.float32),
            pltpu.SemaphoreType.DMA,
            pltpu.SemaphoreType.DMA,
            pltpu.SemaphoreType.DMA,
            pltpu.SemaphoreType.DMA,
            pltpu.SemaphoreType.DMA,
            pltpu.SemaphoreType.DMA,
        ],
    )
    def scpass(src_h, dst_h, stab_h, dtab_h, gin_h, prm_h, gout_h, acc_h,
               idx_s0, idx_s1, idx_d0, idx_d1, srows0, srows1, drows0,
               drows1, gbuf0, gbuf1, gobuf0, gobuf1, obuf, prm_v, zb,
               shacc,
               sem_i0, sem_i1, sem_g0, sem_g1, sem_t0, sem_t1):
        cid = lax.axis_index("c")
        sid = lax.axis_index("s")
        wid = sid * NC + cid
        base = wid * EP
        IS = [idx_s0, idx_s1]
        ID = [idx_d0, idx_d1]
        SR = [srows0, srows1]
        DR = [drows0, drows1]
        GB = [gbuf0, gbuf1]
        GO = [gobuf0, gobuf1]
        SI = [sem_i0, sem_i1]
        SG = [sem_g0, sem_g1]
        ST = [sem_t0, sem_t1]

        pltpu.sync_copy(prm_h, prm_v)
        pv = prm_v[pl.ds(0, L)]
        mn = pv[0]
        gscale = pv[1]

        for c5 in range(5):
            _zero_lane16(zb, 125, c5 * L)
        for j in range(NRC):
            pltpu.sync_copy(zb, shacc.at[pl.ds(sid * NR + j * 125, 125)])
        _zero_lane16(obuf, CH, 64)
        plsc.subcore_barrier()

        def issue_idx(c, s):
            eb = base + c * CH
            pltpu.async_copy(src_h.at[pl.ds(eb, CH)], IS[s], SI[s])
            pltpu.async_copy(dst_h.at[pl.ds(eb, CH)], ID[s], SI[s])
            pltpu.async_copy(gin_h.at[pl.ds(eb, CH)], GB[s], SI[s])

        def drain_idx(c, s):
            eb = base + c * CH
            pltpu.make_async_copy(src_h.at[pl.ds(eb, CH)], IS[s],
                                  SI[s]).wait()
            pltpu.make_async_copy(dst_h.at[pl.ds(eb, CH)], ID[s],
                                  SI[s]).wait()
            pltpu.make_async_copy(gin_h.at[pl.ds(eb, CH)], GB[s],
                                  SI[s]).wait()

        def issue_gather(s):
            pltpu.async_copy(stab_h.at[IS[s]], SR[s], SG[s])
            pltpu.async_copy(dtab_h.at[ID[s]], DR[s], SG[s])

        def drain_gather(s):
            pltpu.make_async_copy(stab_h.at[IS[s]], SR[s], SG[s]).wait()
            pltpu.make_async_copy(dtab_h.at[ID[s]], DR[s], SG[s]).wait()

        def issue_gate(c, s):
            eb = base + c * CH
            pltpu.async_copy(GO[s], gout_h.at[pl.ds(eb, CH)], ST[s])

        def drain_gate(c, s):
            eb = base + c * CH
            pltpu.make_async_copy(GO[s], gout_h.at[pl.ds(eb, CH)],
                                  ST[s]).wait()

        def compute(s):
            def grp(g, _):
                rows = lax.iota(jnp.int32, L) + g * L
                sc = GB[s][pl.ds(g * L, L)]
                gate = (sc - mn) * gscale
                if store_gate:
                    GO[s][pl.ds(g * L, L)] = gate
                for hh in range(heads):
                    elh = plsc.load_gather(SR[s], [rows, _col(32 + hh)])
                    if heads == 1:
                        erh = plsc.load_gather(DR[s], [rows, _col(0)])
                    else:
                        erh = plsc.load_gather(DR[s], [rows, _col(8 + hh)])
                    e = _lrelu(elh + erh)
                    ex = jnp.exp(e * gate)
                    w = 32 // heads
                    for wc in range(hh * w, (hh + 1) * w):
                        wv = plsc.load_gather(SR[s], [rows, _col(wc)])
                        bv = plsc.bitcast(wv, jnp.bfloat16)
                        za, zb2 = plsc.unpack(
                            bv, format=plsc.PackFormat.INTERLEAVED)
                        plsc.store_scatter(obuf, [rows, _col(2 * wc)],
                                           ex * za)
                        plsc.store_scatter(obuf, [rows, _col(2 * wc + 1)],
                                           ex * zb2)
                    plsc.store_scatter(obuf, [rows, _col(64 + hh)], ex)
                return 0

            lax.fori_loop(0, CH // L, grp, 0)

        def chunk(c, b, first, n1, n2):
            drain_gather(b)
            if n1:
                drain_idx(c + 1, 1 - b)
                issue_gather(1 - b)
            if store_gate and not first:
                drain_gate(c - 2, b)
            compute(b)
            pltpu.sync_copy(obuf, shacc.at[ID[b]], add=True)
            if store_gate:
                issue_gate(c, b)
            if n2 == 'always':
                issue_idx(c + 2, b)
            elif n2 == 'cond':
                @pl.when(c + 2 < NCHUNK)
                def _():
                    issue_idx(c + 2, b)

        issue_idx(0, 0)
        drain_idx(0, 0)
        issue_gather(0)
        issue_idx(1, 1)
        chunk(0, 0, True, True, 'always')
        chunk(1, 1, True, True, 'always')

        def pair(j, _):
            c0 = 2 * j
            chunk(c0, 0, False, True, 'always')
            chunk(c0 + 1, 1, False, True, 'cond')
            return 0

        lax.fori_loop(1, (NCHUNK - 1) // 2, pair, 0)
        chunk(NCHUNK - 1, 0, False, False, 'no')
        if store_gate:
            drain_gate(NCHUNK - 2, 1)
            drain_gate(NCHUNK - 1, 0)

        plsc.subcore_barrier()
        for j in range(NRC):
            r0 = sid * NR + j * 125
            pltpu.sync_copy(shacc.at[pl.ds(r0, 125)],
                            acc_h.at[cid, pl.ds(r0, 125)])

    return scpass


def _make_sc4(E, n):
    EP = E // NW
    NCHUNK = EP // CH

    @functools.partial(
        pl.kernel,
        mesh=_mesh(),
        compiler_params=pltpu.CompilerParams(needs_layout_passes=False, use_tc_tiling_on_sc=False),
        out_type=[_f32(E, 2)],
        scratch_types=[
            pltpu.VMEM((CH,), jnp.int32),
            pltpu.VMEM((CH,), jnp.int32),
            pltpu.VMEM((CH,), jnp.int32),
            pltpu.VMEM((CH,), jnp.int32),
            pltpu.VMEM((CH, 16), jnp.float32),
            pltpu.VMEM((CH, 16), jnp.float32),
            pltpu.VMEM((CH, 16), jnp.float32),
            pltpu.VMEM((CH, 16), jnp.float32),
            pltpu.VMEM((CH, 2), jnp.float32),
            pltpu.VMEM((CH, 2), jnp.float32),
            pltpu.VMEM((16,), jnp.float32),
            pltpu.SemaphoreType.DMA,
            pltpu.SemaphoreType.DMA,
            pltpu.SemaphoreType.DMA,
            pltpu.SemaphoreType.DMA,
            pltpu.SemaphoreType.DMA,
            pltpu.SemaphoreType.DMA,
        ],
    )
    def sc4(src_h, dst_h, ptab_h, qtab_h, prm_h, out_h,
            idx_s0, idx_s1, idx_d0, idx_d1, prows0, prows1, qrows0, qrows1,
            obuf0, obuf1, prm_v,
            sem_i0, sem_i1, sem_g0, sem_g1, sem_t0, sem_t1):
        wid = _wid()
        base = wid * EP
        IS = [idx_s0, idx_s1]
        ID = [idx_d0, idx_d1]
        PR = [prows0, prows1]
        QR = [qrows0, qrows1]
        OB = [obuf0, obuf1]
        SI = [sem_i0, sem_i1]
        SG = [sem_g0, sem_g1]
        ST = [sem_t0, sem_t1]
        pltpu.sync_copy(prm_h, prm_v)
        pv = prm_v[pl.ds(0, L)]
        bp0 = pv[0]
        bp1 = pv[1]

        def issue_idx(c, s):
            eb = base + c * CH
            pltpu.async_copy(src_h.at[pl.ds(eb, CH)], IS[s], SI[s])
            pltpu.async_copy(dst_h.at[pl.ds(eb, CH)], ID[s], SI[s])

        def drain_idx(c, s):
            eb = base + c * CH
            pltpu.make_async_copy(src_h.at[pl.ds(eb, CH)], IS[s],
                                  SI[s]).wait()
            pltpu.make_async_copy(dst_h.at[pl.ds(eb, CH)], ID[s],
                                  SI[s]).wait()

        def issue_gather(s):
            pltpu.async_copy(ptab_h.at[IS[s]], PR[s], SG[s])
            pltpu.async_copy(qtab_h.at[ID[s]], QR[s], SG[s])

        def drain_gather(s):
            pltpu.make_async_copy(ptab_h.at[IS[s]], PR[s], SG[s]).wait()
            pltpu.make_async_copy(qtab_h.at[ID[s]], QR[s], SG[s]).wait()

        def issue_store(c, s):
            eb = base + c * CH
            pltpu.async_copy(OB[s], out_h.at[pl.ds(eb, CH)], ST[s])

        def drain_store(c, s):
            eb = base + c * CH
            pltpu.make_async_copy(OB[s], out_h.at[pl.ds(eb, CH)],
                                  ST[s]).wait()

        def chunk(c, b, first, n1, n2):
            drain_gather(b)
            if n1:
                drain_idx(c + 1, 1 - b)
                issue_gather(1 - b)
            if n2 == 'always':
                issue_idx(c + 2, b)
            elif n2 == 'cond':
                @pl.when(c + 2 < NCHUNK)
                def _():
                    issue_idx(c + 2, b)
            if not first:
                drain_store(c - 2, b)
            for g in range(CH // L):
                rows = lax.iota(jnp.int32, L) + g * L
                for cc, bpc in ((0, bp0), (1, bp1)):
                    pc = plsc.load_gather(PR[b], [rows, _col(cc)])
                    qc = plsc.load_gather(QR[b], [rows, _col(cc)])
                    plsc.store_scatter(OB[b], [rows, _col(cc)],
                                       pc + qc + bpc)
            issue_store(c, b)

        issue_idx(0, 0)
        drain_idx(0, 0)
        issue_gather(0)
        issue_idx(1, 1)
        chunk(0, 0, True, True, 'always')
        chunk(1, 1, True, True, 'always')

        def pair(j, _):
            c0 = 2 * j
            chunk(c0, 0, False, True, 'always')
            chunk(c0 + 1, 1, False, True, 'cond')
            return 0

        lax.fori_loop(1, (NCHUNK - 1) // 2, pair, 0)
        chunk(NCHUNK - 1, 0, False, False, 'no')
        drain_store(NCHUNK - 2, 1)
        drain_store(NCHUNK - 1, 0)

    return sc4


def kernel(h, edge_index, W1, b1, W2, b2, W3, b3, fc1, attn1, fc2, attn2,
           Wp, bp):
    n, d = h.shape
    E = edge_index.shape[1]
    nh, _, hd = fc1.shape
    src = edge_index[0]
    dst = edge_index[1]

    f1 = jnp.transpose(fc1, (1, 0, 2)).reshape(d, nh * hd)
    eye = jnp.eye(nh, dtype=jnp.float32)
    al = (attn1[:, :hd, 0][:, :, None] * eye[:, None, :]).reshape(nh * hd, nh)
    ar = (attn1[:, hd:, 0][:, :, None] * eye[:, None, :]).reshape(nh * hd, nh)
    w2a = W2[:16]
    w2b = W2[16:]

    utab, dtab, el, zbf = pl.pallas_call(
        _tc0,
        out_shape=[_f32(n, 16), _f32(n, 16), _f32(n, 4),
                   jax.ShapeDtypeStruct((n, 64), jnp.bfloat16)],
    )(h, W1, b1, w2a, w2b, f1, al, ar)
    zpk = jax.lax.bitcast_convert_type(zbf.reshape(n, 32, 2), jnp.float32)
    s1tab = jnp.concatenate([zpk, el, jnp.zeros((n, 12), jnp.float32)],
                            axis=1)

    prm1 = jnp.concatenate([b2, W3[:, 0], b3,
                            jnp.zeros((15,), jnp.float32)])

    score, mm = _make_sc1(E, n)(src, dst, utab, dtab, prm1)
    mn = jnp.min(mm[:, 0])
    mx = jnp.max(mm[:, 1])
    gscale = 1.0 / (mx - mn)
    prm2 = jnp.zeros((16,), jnp.float32).at[0].set(mn).at[1].set(gscale)

    gate, acc1 = _make_gat_pass(E, n, nh, True)(src, dst, s1tab, dtab,
                                               score, prm2)

    el2, z2bf, d2tab = pl.pallas_call(
        _tc1,
        out_shape=[_f32(n, 1), jax.ShapeDtypeStruct((n, 64), jnp.bfloat16),
                   _f32(n, 16)],
    )(acc1, fc2, attn2[:64], attn2[64:])
    z2pk = jax.lax.bitcast_convert_type(z2bf.reshape(n, 32, 2), jnp.float32)
    s2tab = jnp.concatenate([z2pk, el2, jnp.zeros((n, 15), jnp.float32)],
                            axis=1)

    _, acc2 = _make_gat_pass(E, n, 1, False)(src, dst, s2tab, d2tab,
                                             score, prm2)

    ptab, qtab = pl.pallas_call(
        _tc2,
        out_shape=[_f32(n, 16), _f32(n, 16)],
    )(acc2, Wp[:64], Wp[64:])
    prm4 = jnp.zeros((16,), jnp.float32).at[0].set(bp[0]).at[1].set(bp[1])

    (escore,) = _make_sc4(E, n)(src, dst, ptab, qtab, prm4)

    return escore, gate[:, None]
